# baseline (speedup 1.0000x reference)
import functools

import jax
import jax.numpy as jnp
from jax.experimental import pallas as pl
from jax.experimental.pallas import tpu as pltpu

HIDDEN = 256
HEADS = 4
DH = HIDDEN // HEADS
NUM_MEM = 8
B, Q, K = 2, 128, 256
N = B * Q
BN = 64


def _elu1(x):
    return jnp.where(x > 0, x + 1.0, jnp.exp(x))


def _fused_kernel(qf_ref, kvf_ref, wc_ref, wq_ref, wk_ref, wv_ref, wg_ref,
                  wo_ref, out_ref, router_ref,
                  condk_s, condv_s, condg_s, phiq_s, kvk_s, kvv_s, kvgt_s):
    i = pl.program_id(0)

    @pl.when(i == 0)
    def _projections():
        qf = qf_ref[...]
        kvf = kvf_ref[...]
        cond = jnp.dot(qf, wc_ref[...], preferred_element_type=jnp.float32)
        condk_s[...] = jnp.dot(cond, wk_ref[...], preferred_element_type=jnp.float32)
        condv_s[...] = jnp.dot(cond, wv_ref[...], preferred_element_type=jnp.float32)
        condg_s[...] = jnp.dot(cond, wg_ref[...], preferred_element_type=jnp.float32)
        last = jnp.concatenate(
            [jnp.broadcast_to(kvf[(b + 1) * K - 1:(b + 1) * K, :], (Q, HIDDEN))
             for b in range(B)], axis=0)
        phiq_s[...] = _elu1(jnp.dot(last + cond, wq_ref[...],
                                    preferred_element_type=jnp.float32))
        kvk_s[...] = jnp.dot(kvf, wk_ref[...], preferred_element_type=jnp.float32)
        kvv_s[...] = jnp.dot(kvf, wv_ref[...], preferred_element_type=jnp.float32)
        kvg = jnp.dot(kvf, wg_ref[...], preferred_element_type=jnp.float32)
        kvgt_s[...] = kvg.T

    t0 = i * BN
    b = t0 // Q
    condg = condg_s[pl.ds(t0, BN), :]
    kvgt = kvgt_s[:, pl.ds(b * K, K)]

    router_ref[...] = condg[:, None, :] + kvgt.T[None, :, :]

    neg = jnp.float32(-jnp.inf)
    m1 = jnp.full((BN, K), neg, jnp.float32)
    m2 = jnp.full((BN, K), neg, jnp.float32)
    i1 = jnp.zeros((BN, K), jnp.float32)
    i2 = jnp.zeros((BN, K), jnp.float32)
    for e in range(NUM_MEM):
        v = kvgt[e:e + 1, :] + condg[:, e:e + 1]
        gt1 = v > m1
        gt2 = v > m2
        ef = jnp.float32(e)
        i2 = jnp.where(gt1, i1, jnp.where(gt2, ef, i2))
        m2 = jnp.where(gt1, m1, jnp.where(gt2, v, m2))
        i1 = jnp.where(gt1, ef, i1)
        m1 = jnp.where(gt1, v, m1)
    ex = jnp.exp(m2 - m1)
    g1 = 1.0 / (1.0 + ex)
    g2 = 1.0 - g1

    i1L = i1[:, K - 1:K]
    i2L = i2[:, K - 1:K]
    g1L = g1[:, K - 1:K]
    g2L = g2[:, K - 1:K]
    f32 = lambda c: c.astype(jnp.float32)
    g_dot = (g1L * (g1 * f32(i1 == i1L) + g2 * f32(i2 == i1L)) +
             g2L * (g1 * f32(i1 == i2L) + g2 * f32(i2 == i2L)))

    condk = condk_s[pl.ds(t0, BN), :]
    kvk = kvk_s[pl.ds(b * K, K), :]
    kvv = kvv_s[pl.ds(b * K, K), :]
    phiq = phiq_s[pl.ds(t0, BN), :]
    condv = condv_s[pl.ds(t0, BN), :]
    phik = _elu1(condk[:, None, :] + kvk[None, :, :])

    d_idx = jax.lax.broadcasted_iota(jnp.int32, (HIDDEN, HEADS), 0)
    h_idx = jax.lax.broadcasted_iota(jnp.int32, (HIDDEN, HEADS), 1)
    M = (d_idx // DH == h_idx).astype(jnp.float32)

    psi = (phik * phiq[:, None, :] * g_dot[:, :, None]).reshape(BN * K, HIDDEN)
    a_small = jnp.dot(psi, M, preferred_element_type=jnp.float32)
    wgt_exp = jnp.dot(a_small, M.T,
                      preferred_element_type=jnp.float32).reshape(BN, K, HIDDEN)
    den = jnp.sum(wgt_exp, axis=1)
    num = jnp.sum(wgt_exp * kvv[None, :, :], axis=1)
    out_attn = (num + den * condv) / (den + 1e-6)
    out_ref[...] = jnp.dot(out_attn, wo_ref[...],
                           preferred_element_type=jnp.float32)


@functools.partial(jax.jit, static_argnames=("interpret",))
def _run(query, keyval, W_cond, W_q, W_k, W_v, W_g, W_o, interpret=False):
    D = HIDDEN
    qf = query.reshape(N, D)
    kvf = keyval.reshape(B * K, D)
    f32 = jnp.float32
    nblk = N // BN

    full = lambda *shape: pl.BlockSpec(shape, lambda i: tuple(0 for _ in shape))
    out, router = pl.pallas_call(
        _fused_kernel,
        grid=(nblk,),
        in_specs=[
            full(N, D),
            full(B * K, D),
            full(D, D),
            full(D, D),
            full(D, D),
            full(D, D),
            full(D, NUM_MEM),
            full(D, D),
        ],
        out_specs=(
            pl.BlockSpec((BN, D), lambda i: (i, 0)),
            pl.BlockSpec((BN, K, NUM_MEM), lambda i: (i, 0, 0)),
        ),
        out_shape=(
            jax.ShapeDtypeStruct((N, D), f32),
            jax.ShapeDtypeStruct((N, K, NUM_MEM), f32),
        ),
        scratch_shapes=[
            pltpu.VMEM((N, D), f32),
            pltpu.VMEM((N, D), f32),
            pltpu.VMEM((N, NUM_MEM), f32),
            pltpu.VMEM((N, D), f32),
            pltpu.VMEM((B * K, D), f32),
            pltpu.VMEM((B * K, D), f32),
            pltpu.VMEM((NUM_MEM, B * K), f32),
        ],
        interpret=interpret,
    )(qf, kvf, W_cond, W_q, W_k, W_v, W_g, W_o)

    return out.reshape(B, Q, D), router


def kernel(query, keyval, W_cond, W_q, W_k, W_v, W_g, W_o):
    return _run(query, keyval, W_cond, W_q, W_k, W_v, W_g, W_o)


# speedup vs baseline: 1.7131x; 1.7131x over previous
import functools

import jax
import jax.numpy as jnp
from jax.experimental import pallas as pl
from jax.experimental.pallas import tpu as pltpu

HIDDEN = 256
HEADS = 4
DH = HIDDEN // HEADS
NUM_MEM = 8
B, Q, K = 2, 128, 256
N = B * Q
BN = 64


def _elu1(x):
    return jnp.where(x > 0, x + 1.0, jnp.exp(x))


def _fused_kernel(qf_ref, kvf_ref, wc_ref, wq_ref, wk_ref, wv_ref, wg_ref,
                  wo_ref, out_ref, router_ref,
                  condk_s, condv_s, condg_s, phiq_s, kvk_s, kvv_s, kvgt_s):
    i = pl.program_id(0)

    @pl.when(i == 0)
    def _projections():
        qf = qf_ref[...]
        kvf = kvf_ref[...]
        cond = jnp.dot(qf, wc_ref[...], preferred_element_type=jnp.float32)
        condk_s[...] = jnp.dot(cond, wk_ref[...], preferred_element_type=jnp.float32)
        condv_s[...] = jnp.dot(cond, wv_ref[...], preferred_element_type=jnp.float32)
        condg_s[...] = jnp.dot(cond, wg_ref[...], preferred_element_type=jnp.float32)
        last = jnp.concatenate(
            [jnp.broadcast_to(kvf[(b + 1) * K - 1:(b + 1) * K, :], (Q, HIDDEN))
             for b in range(B)], axis=0)
        phiq_s[...] = _elu1(jnp.dot(last + cond, wq_ref[...],
                                    preferred_element_type=jnp.float32))
        kvk_s[...] = jnp.dot(kvf, wk_ref[...], preferred_element_type=jnp.float32)
        kvv_s[...] = jnp.dot(kvf, wv_ref[...], preferred_element_type=jnp.float32)
        kvg = jnp.dot(kvf, wg_ref[...], preferred_element_type=jnp.float32)
        kvgt_s[...] = kvg.T

    t0 = i * BN
    b = t0 // Q
    condg = condg_s[pl.ds(t0, BN), :]
    kvgt = kvgt_s[:, pl.ds(b * K, K)]

    router_ref[...] = condg[:, :, None] + kvgt[None, :, :]

    neg = jnp.float32(-jnp.inf)
    m1 = jnp.full((BN, K), neg, jnp.float32)
    m2 = jnp.full((BN, K), neg, jnp.float32)
    i1 = jnp.zeros((BN, K), jnp.float32)
    i2 = jnp.zeros((BN, K), jnp.float32)
    for e in range(NUM_MEM):
        v = kvgt[e:e + 1, :] + condg[:, e:e + 1]
        gt1 = v > m1
        gt2 = v > m2
        ef = jnp.float32(e)
        i2 = jnp.where(gt1, i1, jnp.where(gt2, ef, i2))
        m2 = jnp.where(gt1, m1, jnp.where(gt2, v, m2))
        i1 = jnp.where(gt1, ef, i1)
        m1 = jnp.where(gt1, v, m1)
    ex = jnp.exp(m2 - m1)
    g1 = 1.0 / (1.0 + ex)
    g2 = 1.0 - g1

    i1L = i1[:, K - 1:K]
    i2L = i2[:, K - 1:K]
    g1L = g1[:, K - 1:K]
    g2L = g2[:, K - 1:K]
    f32 = lambda c: c.astype(jnp.float32)
    g_dot = (g1L * (g1 * f32(i1 == i1L) + g2 * f32(i2 == i1L)) +
             g2L * (g1 * f32(i1 == i2L) + g2 * f32(i2 == i2L)))

    condk = condk_s[pl.ds(t0, BN), :]
    kvk = kvk_s[pl.ds(b * K, K), :]
    kvv = kvv_s[pl.ds(b * K, K), :]
    phiq = phiq_s[pl.ds(t0, BN), :]
    condv = condv_s[pl.ds(t0, BN), :]
    phik = _elu1(condk[:, None, :] + kvk[None, :, :])

    d_idx = jax.lax.broadcasted_iota(jnp.int32, (HIDDEN, HEADS), 0)
    h_idx = jax.lax.broadcasted_iota(jnp.int32, (HIDDEN, HEADS), 1)
    M = (d_idx // DH == h_idx).astype(jnp.float32)

    psi = (phik * phiq[:, None, :] * g_dot[:, :, None]).reshape(BN * K, HIDDEN)
    a_small = jnp.dot(psi, M, preferred_element_type=jnp.float32)
    wgt_exp = jnp.dot(a_small, M.T,
                      preferred_element_type=jnp.float32).reshape(BN, K, HIDDEN)
    den = jnp.sum(wgt_exp, axis=1)
    num = jnp.sum(wgt_exp * kvv[None, :, :], axis=1)
    out_attn = (num + den * condv) / (den + 1e-6)
    out_ref[...] = jnp.dot(out_attn, wo_ref[...],
                           preferred_element_type=jnp.float32)


@functools.partial(jax.jit, static_argnames=("interpret",))
def _run(query, keyval, W_cond, W_q, W_k, W_v, W_g, W_o, interpret=False):
    D = HIDDEN
    qf = query.reshape(N, D)
    kvf = keyval.reshape(B * K, D)
    f32 = jnp.float32
    nblk = N // BN

    full = lambda *shape: pl.BlockSpec(shape, lambda i: tuple(0 for _ in shape))
    out, router = pl.pallas_call(
        _fused_kernel,
        grid=(nblk,),
        in_specs=[
            full(N, D),
            full(B * K, D),
            full(D, D),
            full(D, D),
            full(D, D),
            full(D, D),
            full(D, NUM_MEM),
            full(D, D),
        ],
        out_specs=(
            pl.BlockSpec((BN, D), lambda i: (i, 0)),
            pl.BlockSpec((BN, NUM_MEM, K), lambda i: (i, 0, 0)),
        ),
        out_shape=(
            jax.ShapeDtypeStruct((N, D), f32),
            jax.ShapeDtypeStruct((N, NUM_MEM, K), f32),
        ),
        scratch_shapes=[
            pltpu.VMEM((N, D), f32),
            pltpu.VMEM((N, D), f32),
            pltpu.VMEM((N, NUM_MEM), f32),
            pltpu.VMEM((N, D), f32),
            pltpu.VMEM((B * K, D), f32),
            pltpu.VMEM((B * K, D), f32),
            pltpu.VMEM((NUM_MEM, B * K), f32),
        ],
        interpret=interpret,
    )(qf, kvf, W_cond, W_q, W_k, W_v, W_g, W_o)

    return out.reshape(B, Q, D), router.transpose(0, 2, 1)


def kernel(query, keyval, W_cond, W_q, W_k, W_v, W_g, W_o):
    return _run(query, keyval, W_cond, W_q, W_k, W_v, W_g, W_o)
